# baseline (speedup 1.0000x reference)
import functools

import jax
import jax.numpy as jnp
from jax.experimental import pallas as pl
from jax.experimental.pallas import tpu as pltpu

_EPS = 1e-5
_EDGE_BLK = 16000


def _ln(x):
    m = jnp.mean(x, axis=-1, keepdims=True)
    c = x - m
    v = jnp.mean(c * c, axis=-1, keepdims=True)
    return c * jax.lax.rsqrt(v + _EPS)


def _proj(x, w, bias):
    return jnp.dot(x.astype(jnp.bfloat16), w,
                   preferred_element_type=jnp.float32) + bias


def _fused_body(edge_ref, node_ref, q_ref,
                rel_w_ref, rel_b_ref, node_w_ref, node_b_ref,
                q_w_ref, q_b_ref,
                rel_out_ref, node_out_ref, q_out_ref, *, last_step):
    rel_out_ref[:] = _proj(_ln(edge_ref[:]), rel_w_ref[:], rel_b_ref[:])

    @pl.when(pl.program_id(0) == last_step)
    def _():
        node_out_ref[:] = _proj(_ln(node_ref[:]), node_w_ref[:], node_b_ref[:])
        q_out_ref[:] = _proj(q_ref[:], q_w_ref[:], q_b_ref[:])


def kernel(node_embeddings, edge_embeddings, question_emb,
           node_norm_g, node_norm_b, rel_norm_g, rel_norm_b,
           node_W, node_b, rel_W, rel_b, q_W, q_b):
    n_rows, d = node_embeddings.shape
    e_rows, _ = edge_embeddings.shape
    b_rows, _ = question_emb.shape
    h = node_W.shape[1]

    node_Wg = (node_norm_g[:, None] * node_W).astype(jnp.bfloat16)
    node_bias2 = (node_norm_b @ node_W + node_b).reshape(1, h)
    rel_Wg = (rel_norm_g[:, None] * rel_W).astype(jnp.bfloat16)
    rel_bias2 = (rel_norm_b @ rel_W + rel_b).reshape(1, h)

    grid = pl.cdiv(e_rows, _EDGE_BLK)
    const = lambda i: (0, 0)
    body = functools.partial(_fused_body, last_step=grid - 1)

    rel_out, node_out, q_out = pl.pallas_call(
        body,
        grid=(grid,),
        in_specs=[
            pl.BlockSpec((_EDGE_BLK, d), lambda i: (i, 0)),
            pl.BlockSpec((n_rows, d), const),
            pl.BlockSpec((b_rows, d), const),
            pl.BlockSpec((d, h), const),
            pl.BlockSpec((1, h), const),
            pl.BlockSpec((d, h), const),
            pl.BlockSpec((1, h), const),
            pl.BlockSpec((d, h), const),
            pl.BlockSpec((1, h), const),
        ],
        out_specs=[
            pl.BlockSpec((_EDGE_BLK, h), lambda i: (i, 0)),
            pl.BlockSpec((n_rows, h), const),
            pl.BlockSpec((b_rows, h), const),
        ],
        out_shape=[
            jax.ShapeDtypeStruct((e_rows, h), jnp.float32),
            jax.ShapeDtypeStruct((n_rows, h), jnp.float32),
            jax.ShapeDtypeStruct((b_rows, h), jnp.float32),
        ],
        compiler_params=pltpu.CompilerParams(
            dimension_semantics=("arbitrary",)),
    )(edge_embeddings, node_embeddings, question_emb,
      rel_Wg, rel_bias2, node_Wg, node_bias2,
      q_W.astype(jnp.bfloat16), q_b.reshape(1, h))

    return (node_out, rel_out, q_out)
